# baseline (speedup 1.0000x reference)
import functools

import jax
import jax.numpy as jnp
from jax import lax
from jax.experimental import pallas as pl
from jax.experimental.pallas import tpu as pltpu
from jax.experimental.pallas import tpu_sc as plsc

N = 10000
NP = 10240
E = 320000
D = 128
EPS = 1e-5

NC = 2
NS = 16
NW = NC * NS
CHUNK = 128
EP = 327680
EROWS = EP // CHUNK
WROWS = EROWS // NW
RPT = NP // NS
ZR = 32
CNT_CHUNK = 2048


def _sc_agg_body(with_cnt, *refs):
    if with_cnt:
        (y_hbm, src_hbm, dst_hbm, z2d_hbm, z1d_hbm, agg_out, cnt_out,
         src_v, dst_v, rows0_v, rows1_v, ones_v,
         acc_sh, cnt_sh, gsem0, gsem1) = refs
    else:
        (y_hbm, src_hbm, dst_hbm, z2d_hbm, agg_out,
         src_v, dst_v, rows0_v, rows1_v,
         acc_sh, gsem0, gsem1) = refs
    rows = (rows0_v, rows1_v)
    gsems = (gsem0, gsem1)
    HW = WROWS // 2

    cid = lax.axis_index("c")
    sid = lax.axis_index("s")
    wid = sid * NC + cid

    pltpu.sync_copy(src_hbm.at[pl.ds(wid * WROWS, WROWS)], src_v)
    pltpu.sync_copy(dst_hbm.at[pl.ds(wid * WROWS, HW)], dst_v)

    pltpu.sync_copy(z2d_hbm, acc_sh.at[pl.ds(sid * RPT, RPT)])

    if with_cnt:
        for j in range(CHUNK // 16):
            ones_v[pl.ds(j * 16, 16)] = jnp.ones((16,), jnp.float32)
        pltpu.sync_copy(z1d_hbm, cnt_sh.at[pl.ds(sid * RPT, RPT)])

    plsc.subcore_barrier()

    pltpu.async_copy(y_hbm.at[src_v.at[0]], rows0_v, gsem0)
    pltpu.async_copy(y_hbm.at[src_v.at[1]], rows1_v, gsem1)

    def make_edge_body(seg):
        def edge_body(p, _):
            for b in range(2):
                g = p * 2 + b
                d = g - seg * HW
                pltpu.make_async_copy(y_hbm.at[src_v.at[g]], rows[b], gsems[b]).wait()
                pltpu.sync_copy(rows[b], acc_sh.at[dst_v.at[d]], add=True)
                if with_cnt:
                    pltpu.sync_copy(ones_v, cnt_sh.at[dst_v.at[d]], add=True)

                @pl.when(g + 2 < WROWS)
                def _():
                    pltpu.async_copy(y_hbm.at[src_v.at[g + 2]], rows[b], gsems[b])
            return 0
        return edge_body

    lax.fori_loop(0, HW // 2, make_edge_body(0), 0)
    pltpu.sync_copy(dst_hbm.at[pl.ds(wid * WROWS + HW, HW)], dst_v)
    lax.fori_loop(HW // 2, WROWS // 2, make_edge_body(1), 0)

    plsc.subcore_barrier()

    pltpu.sync_copy(acc_sh.at[pl.ds(sid * RPT, RPT)],
                    agg_out.at[cid, pl.ds(sid * RPT, RPT)])
    if with_cnt:
        pltpu.sync_copy(cnt_sh.at[pl.ds(sid * RPT, RPT)],
                        cnt_out.at[cid, pl.ds(sid * RPT, RPT)])


@functools.lru_cache(maxsize=None)
def _make_sc_agg(with_cnt):
    mesh = plsc.VectorSubcoreMesh(core_axis_name="c", subcore_axis_name="s",
                                  num_cores=NC, num_subcores=NS)
    out_type = [jax.ShapeDtypeStruct((NC, NP, D), jnp.float32)]
    scratch = [
        pltpu.VMEM((WROWS, CHUNK), jnp.int32),
        pltpu.VMEM((WROWS // 2, CHUNK), jnp.int32),
        pltpu.VMEM((CHUNK, D), jnp.float32),
        pltpu.VMEM((CHUNK, D), jnp.float32),
    ]
    if with_cnt:
        out_type.append(jax.ShapeDtypeStruct((NC, NP), jnp.float32))
        scratch += [
            pltpu.VMEM((CHUNK,), jnp.float32),
        ]
    scratch.append(pltpu.VMEM_SHARED((NP, D), jnp.float32))
    if with_cnt:
        scratch.append(pltpu.VMEM_SHARED((NP,), jnp.float32))
    scratch += [pltpu.SemaphoreType.DMA, pltpu.SemaphoreType.DMA]

    return pl.kernel(
        functools.partial(_sc_agg_body, with_cnt),
        out_type=out_type,
        mesh=mesh,
        scratch_types=scratch,
    )


R = 1024
_G = NP // R


def _tc_pre_body(x_ref, wl_ref, wr_ref, bl_ref, y_out, z_out):
    h = x_ref[...]
    y_out[...] = jnp.dot(h, wl_ref[...], preferred_element_type=jnp.float32)
    z_out[...] = jnp.dot(h, wr_ref[...], preferred_element_type=jnp.float32) + bl_ref[...]


def _combine_mean(agg_ref, cnt_ref, z_ref):
    i = pl.program_id(0)
    s = agg_ref[0] + agg_ref[1]
    c = cnt_ref[0, pl.ds(i * R, R)] + cnt_ref[1, pl.ds(i * R, R)]
    inv = 1.0 / jnp.clip(c, 1.0, None)
    return s * inv[:, None] + z_ref[...]


def _tc_mid_body(agg_ref, cnt_ref, z_ref, wl_ref, wr_ref, bl_ref, g_ref, b_ref,
                 y_out, z_out):
    pre = _combine_mean(agg_ref, cnt_ref, z_ref)
    h = jnp.maximum(pre, 0.0)
    mu = jnp.mean(h, axis=-1, keepdims=True)
    var = jnp.mean((h - mu) ** 2, axis=-1, keepdims=True)
    hn = (h - mu) * lax.rsqrt(var + EPS) * g_ref[...] + b_ref[...]
    y_out[...] = jnp.dot(hn, wl_ref[...], preferred_element_type=jnp.float32)
    z_out[...] = jnp.dot(hn, wr_ref[...], preferred_element_type=jnp.float32) + bl_ref[...]


def _tc_post_body(agg_ref, cnt_ref, z_ref, out_ref):
    out_ref[...] = _combine_mean(agg_ref, cnt_ref, z_ref)


_row_spec = pl.BlockSpec((R, D), lambda i: (i, 0))
_w_spec = pl.BlockSpec((D, D), lambda i: (0, 0))
_b_spec = pl.BlockSpec((1, D), lambda i: (0, 0))
_agg_spec = pl.BlockSpec((NC, R, D), lambda i: (0, i, 0))
_cnt_spec = pl.BlockSpec((NC, NP), lambda i: (0, 0))

_tc_pre = pl.pallas_call(
    _tc_pre_body,
    grid=(_G,),
    in_specs=[_row_spec, _w_spec, _w_spec, _b_spec],
    out_specs=[_row_spec, _row_spec],
    out_shape=[jax.ShapeDtypeStruct((NP, D), jnp.float32)] * 2,
)

_tc_mid = pl.pallas_call(
    _tc_mid_body,
    grid=(_G,),
    in_specs=[_agg_spec, _cnt_spec, _row_spec, _w_spec, _w_spec,
              _b_spec, _b_spec, _b_spec],
    out_specs=[_row_spec, _row_spec],
    out_shape=[jax.ShapeDtypeStruct((NP, D), jnp.float32)] * 2,
)

_tc_post = pl.pallas_call(
    _tc_post_body,
    grid=(_G,),
    in_specs=[_agg_spec, _cnt_spec, _row_spec],
    out_specs=_row_spec,
    out_shape=jax.ShapeDtypeStruct((NP, D), jnp.float32),
)


def kernel(x, edge_index, Wl0, bl0, Wr0, Wl1, bl1, Wr1, Wl2, bl2, Wr2, ln_g, ln_b):
    pad_iota = jnp.arange(EP - E, dtype=jnp.int32)
    src = jnp.concatenate([edge_index[0], pad_iota % N]).reshape(EROWS, CHUNK)
    dst = jnp.concatenate([edge_index[1],
                           N + pad_iota % (NP - N)]).reshape(EROWS, CHUNK)
    xp = jnp.pad(x, ((0, NP - N), (0, 0)))
    bl0_2 = bl0.reshape(1, D)
    bl1_2 = bl1.reshape(1, D)
    bl2_2 = bl2.reshape(1, D)
    g2 = ln_g.reshape(1, D)
    b2 = ln_b.reshape(1, D)

    sc_agg_cnt = _make_sc_agg(True)
    sc_agg = _make_sc_agg(False)
    z2d = jnp.zeros((RPT, D), jnp.float32)
    z1d = jnp.zeros((RPT,), jnp.float32)

    y0, z0 = _tc_pre(xp, Wl0.T, Wr0.T, bl0_2)
    agg0, cnt = sc_agg_cnt(y0, src, dst, z2d, z1d)
    y1, z1 = _tc_mid(agg0, cnt, z0, Wl1.T, Wr1.T, bl1_2, g2, b2)
    (agg1,) = sc_agg(y1, src, dst, z2d)
    y2, z2 = _tc_mid(agg1, cnt, z1, Wl2.T, Wr2.T, bl2_2, g2, b2)
    (agg2,) = sc_agg(y2, src, dst, z2d)
    out = _tc_post(agg2, cnt, z2)
    return out[:N]


# speedup vs baseline: 1.0699x; 1.0699x over previous
import functools

import jax
import jax.numpy as jnp
from jax import lax
from jax.experimental import pallas as pl
from jax.experimental.pallas import tpu as pltpu
from jax.experimental.pallas import tpu_sc as plsc

N = 10000
NP = 10240
E = 320000
D = 128
EPS = 1e-5

NC = 2
NS = 16
NW = NC * NS
CHUNK = 128
EP = 327680
EROWS = EP // CHUNK
WROWS = EROWS // NW
RPT = NP // NS
ZR = 32
CNT_CHUNK = 2048


def _sc_agg_body(with_cnt, *refs):
    if with_cnt:
        (y_hbm, src_hbm, dst_hbm, z2d_hbm, z1d_hbm, agg_out, cnt_out,
         src_v, dst_v, rows0_v, rows1_v, ones_v,
         acc_sh, cnt_sh, gsem0, gsem1) = refs
    else:
        (y_hbm, src_hbm, dst_hbm, z2d_hbm, agg_out,
         src_v, dst_v, rows0_v, rows1_v,
         acc_sh, gsem0, gsem1) = refs
    rows = (rows0_v, rows1_v)
    gsems = (gsem0, gsem1)
    HW = WROWS // 2

    cid = lax.axis_index("c")
    sid = lax.axis_index("s")
    wid = sid * NC + cid

    pltpu.sync_copy(src_hbm.at[pl.ds(wid * WROWS, WROWS)], src_v)
    pltpu.sync_copy(dst_hbm.at[pl.ds(wid * WROWS, HW)], dst_v)

    pltpu.sync_copy(z2d_hbm, acc_sh.at[pl.ds(sid * RPT, RPT)])

    if with_cnt:
        for j in range(CHUNK // 16):
            ones_v[pl.ds(j * 16, 16)] = jnp.ones((16,), jnp.float32)
        pltpu.sync_copy(z1d_hbm, cnt_sh.at[pl.ds(sid * RPT, RPT)])

    plsc.subcore_barrier()

    pltpu.async_copy(y_hbm.at[src_v.at[0]], rows0_v, gsem0)
    pltpu.async_copy(y_hbm.at[src_v.at[1]], rows1_v, gsem1)

    def make_edge_body(seg):
        def edge_body(p, _):
            for b in range(2):
                g = p * 2 + b
                d = g - seg * HW
                pltpu.make_async_copy(y_hbm.at[src_v.at[g]], rows[b], gsems[b]).wait()
                pltpu.sync_copy(rows[b], acc_sh.at[dst_v.at[d]], add=True)
                if with_cnt:
                    pltpu.sync_copy(ones_v, cnt_sh.at[dst_v.at[d]], add=True)

                @pl.when(g + 2 < WROWS)
                def _():
                    pltpu.async_copy(y_hbm.at[src_v.at[g + 2]], rows[b], gsems[b])
            return 0
        return edge_body

    lax.fori_loop(0, HW // 2, make_edge_body(0), 0)
    pltpu.sync_copy(dst_hbm.at[pl.ds(wid * WROWS + HW, HW)], dst_v)
    lax.fori_loop(HW // 2, WROWS // 2, make_edge_body(1), 0)

    plsc.subcore_barrier()

    pltpu.sync_copy(acc_sh.at[pl.ds(sid * RPT, RPT)],
                    agg_out.at[cid, pl.ds(sid * RPT, RPT)])
    if with_cnt:
        pltpu.sync_copy(cnt_sh.at[pl.ds(sid * RPT, RPT)],
                        cnt_out.at[cid, pl.ds(sid * RPT, RPT)])


@functools.lru_cache(maxsize=None)
def _make_sc_agg(with_cnt):
    mesh = plsc.VectorSubcoreMesh(core_axis_name="c", subcore_axis_name="s",
                                  num_cores=NC, num_subcores=NS)
    out_type = [jax.ShapeDtypeStruct((NC, NP, D), jnp.float32)]
    scratch = [
        pltpu.VMEM((WROWS, CHUNK), jnp.int32),
        pltpu.VMEM((WROWS // 2, CHUNK), jnp.int32),
        pltpu.VMEM((CHUNK, D), jnp.float32),
        pltpu.VMEM((CHUNK, D), jnp.float32),
    ]
    if with_cnt:
        out_type.append(jax.ShapeDtypeStruct((NC, NP), jnp.float32))
        scratch += [
            pltpu.VMEM((CHUNK,), jnp.float32),
        ]
    scratch.append(pltpu.VMEM_SHARED((NP, D), jnp.float32))
    if with_cnt:
        scratch.append(pltpu.VMEM_SHARED((NP,), jnp.float32))
    scratch += [pltpu.SemaphoreType.DMA, pltpu.SemaphoreType.DMA]

    return pl.kernel(
        functools.partial(_sc_agg_body, with_cnt),
        out_type=out_type,
        mesh=mesh,
        scratch_types=scratch,
    )


R = 1024
_G = (N + R - 1) // R


def _combine_mean(agg_ref, cnt_ref):
    i = pl.program_id(0)
    s = agg_ref[0] + agg_ref[1]
    c = cnt_ref[0, pl.ds(i * R, R)] + cnt_ref[1, pl.ds(i * R, R)]
    inv = 1.0 / jnp.clip(c, 1.0, None)
    return s * inv[:, None]


def _layer_out(agg_ref, cnt_ref, h_ref, wl_ref, wr_ref, bl_ref):
    mean = _combine_mean(agg_ref, cnt_ref)
    return (jnp.dot(mean, wl_ref[...], preferred_element_type=jnp.float32)
            + jnp.dot(h_ref[...], wr_ref[...], preferred_element_type=jnp.float32)
            + bl_ref[...])


def _tc_mid_body(agg_ref, cnt_ref, h_ref, wl_ref, wr_ref, bl_ref, g_ref, b_ref,
                 h_out):
    pre = _layer_out(agg_ref, cnt_ref, h_ref, wl_ref, wr_ref, bl_ref)
    h = jnp.maximum(pre, 0.0)
    mu = jnp.mean(h, axis=-1, keepdims=True)
    var = jnp.mean((h - mu) ** 2, axis=-1, keepdims=True)
    h_out[...] = (h - mu) * lax.rsqrt(var + EPS) * g_ref[...] + b_ref[...]


def _tc_post_body(agg_ref, cnt_ref, h_ref, wl_ref, wr_ref, bl_ref, out_ref):
    out_ref[...] = _layer_out(agg_ref, cnt_ref, h_ref, wl_ref, wr_ref, bl_ref)


_row_spec = pl.BlockSpec((R, D), lambda i: (i, 0))
_w_spec = pl.BlockSpec((D, D), lambda i: (0, 0))
_b_spec = pl.BlockSpec((1, D), lambda i: (0, 0))
_agg_spec = pl.BlockSpec((NC, R, D), lambda i: (0, i, 0))
_cnt_spec = pl.BlockSpec((NC, NP), lambda i: (0, 0))

_tc_mid = pl.pallas_call(
    _tc_mid_body,
    grid=(_G,),
    in_specs=[_agg_spec, _cnt_spec, _row_spec, _w_spec, _w_spec,
              _b_spec, _b_spec, _b_spec],
    out_specs=_row_spec,
    out_shape=jax.ShapeDtypeStruct((N, D), jnp.float32),
)

_tc_post = pl.pallas_call(
    _tc_post_body,
    grid=(_G,),
    in_specs=[_agg_spec, _cnt_spec, _row_spec, _w_spec, _w_spec, _b_spec],
    out_specs=_row_spec,
    out_shape=jax.ShapeDtypeStruct((N, D), jnp.float32),
)


def kernel(x, edge_index, Wl0, bl0, Wr0, Wl1, bl1, Wr1, Wl2, bl2, Wr2, ln_g, ln_b):
    pad_iota = jnp.arange(EP - E, dtype=jnp.int32)
    src = jnp.concatenate([edge_index[0], pad_iota]).reshape(EROWS, CHUNK)
    dst = jnp.concatenate([edge_index[1],
                           N + (pad_iota & 127)]).reshape(EROWS, CHUNK)
    bl0_2 = bl0.reshape(1, D)
    bl1_2 = bl1.reshape(1, D)
    bl2_2 = bl2.reshape(1, D)
    g2 = ln_g.reshape(1, D)
    b2 = ln_b.reshape(1, D)

    sc_agg_cnt = _make_sc_agg(True)
    sc_agg = _make_sc_agg(False)
    z2d = jnp.zeros((RPT, D), jnp.float32)
    z1d = jnp.zeros((RPT,), jnp.float32)

    agg0, cnt = sc_agg_cnt(x, src, dst, z2d, z1d)
    h1 = _tc_mid(agg0, cnt, x, Wl0.T, Wr0.T, bl0_2, g2, b2)
    (agg1,) = sc_agg(h1, src, dst, z2d)
    h2 = _tc_mid(agg1, cnt, h1, Wl1.T, Wr1.T, bl1_2, g2, b2)
    (agg2,) = sc_agg(h2, src, dst, z2d)
    return _tc_post(agg2, cnt, h2, Wl2.T, Wr2.T, bl2_2)


# speedup vs baseline: 1.0984x; 1.0267x over previous
import functools

import jax
import jax.numpy as jnp
from jax import lax
from jax.experimental import pallas as pl
from jax.experimental.pallas import tpu as pltpu
from jax.experimental.pallas import tpu_sc as plsc

N = 10000
NP = 10240
E = 320000
D = 128
EPS = 1e-5

NC = 2
NS = 16
NW = NC * NS
CHUNK = 128
EP = 327680
EROWS = EP // CHUNK
WROWS = EROWS // NW
RPT = NP // NS
ZR = 32
CNT_CHUNK = 2048


def _sc_agg_body(with_cnt, *refs):
    if with_cnt:
        (y_hbm, src_hbm, dst_hbm, agg_out, cnt_out,
         src_v, dst_v, rows0_v, rows1_v, ones_v,
         acc_sh, cnt_sh, gsem0, gsem1) = refs
    else:
        (y_hbm, src_hbm, dst_hbm, agg_out,
         src_v, dst_v, rows0_v, rows1_v,
         acc_sh, gsem0, gsem1) = refs
    rows = (rows0_v, rows1_v)
    gsems = (gsem0, gsem1)
    HW = WROWS // 2

    cid = lax.axis_index("c")
    sid = lax.axis_index("s")
    wid = sid * NC + cid

    pltpu.sync_copy(src_hbm.at[pl.ds(wid * WROWS, WROWS)], src_v)
    pltpu.sync_copy(dst_hbm.at[pl.ds(wid * WROWS, HW)], dst_v)

    zvec = jnp.zeros((16,), jnp.float32)
    for r in range(ZR):
        for j in range(D // 16):
            rows0_v[r, pl.ds(j * 16, 16)] = zvec

    def zero_body(g, _):
        pltpu.sync_copy(rows0_v.at[pl.ds(0, ZR)],
                        acc_sh.at[pl.ds(sid * RPT + g * ZR, ZR)])
        return 0
    lax.fori_loop(0, RPT // ZR, zero_body, 0)

    if with_cnt:
        for j in range(CHUNK // 16):
            ones_v[pl.ds(j * 16, 16)] = jnp.ones((16,), jnp.float32)

        def zero_cnt(g, _):
            pltpu.sync_copy(rows0_v.at[0],
                            cnt_sh.at[pl.ds(sid * RPT + g * D, D)])
            return 0
        lax.fori_loop(0, RPT // D, zero_cnt, 0)

    plsc.subcore_barrier()

    pltpu.async_copy(y_hbm.at[src_v.at[0]], rows0_v, gsem0)
    pltpu.async_copy(y_hbm.at[src_v.at[1]], rows1_v, gsem1)

    def make_edge_body(seg):
        def edge_body(p, _):
            for b in range(2):
                g = p * 2 + b
                d = g - seg * HW
                pltpu.make_async_copy(y_hbm.at[src_v.at[g]], rows[b], gsems[b]).wait()
                pltpu.sync_copy(rows[b], acc_sh.at[dst_v.at[d]], add=True)
                if with_cnt:
                    pltpu.sync_copy(ones_v, cnt_sh.at[dst_v.at[d]], add=True)

                @pl.when(g + 2 < WROWS)
                def _():
                    pltpu.async_copy(y_hbm.at[src_v.at[g + 2]], rows[b], gsems[b])
            return 0
        return edge_body

    lax.fori_loop(0, HW // 2, make_edge_body(0), 0)
    pltpu.sync_copy(dst_hbm.at[pl.ds(wid * WROWS + HW, HW)], dst_v)
    lax.fori_loop(HW // 2, WROWS // 2, make_edge_body(1), 0)

    plsc.subcore_barrier()

    pltpu.sync_copy(acc_sh.at[pl.ds(sid * RPT, RPT)],
                    agg_out.at[cid, pl.ds(sid * RPT, RPT)])
    if with_cnt:
        pltpu.sync_copy(cnt_sh.at[pl.ds(sid * RPT, RPT)],
                        cnt_out.at[cid, pl.ds(sid * RPT, RPT)])


@functools.lru_cache(maxsize=None)
def _make_sc_agg(with_cnt):
    mesh = plsc.VectorSubcoreMesh(core_axis_name="c", subcore_axis_name="s",
                                  num_cores=NC, num_subcores=NS)
    out_type = [jax.ShapeDtypeStruct((NC, NP, D), jnp.float32)]
    scratch = [
        pltpu.VMEM((WROWS, CHUNK), jnp.int32),
        pltpu.VMEM((WROWS // 2, CHUNK), jnp.int32),
        pltpu.VMEM((CHUNK, D), jnp.float32),
        pltpu.VMEM((CHUNK, D), jnp.float32),
    ]
    if with_cnt:
        out_type.append(jax.ShapeDtypeStruct((NC, NP), jnp.float32))
        scratch += [
            pltpu.VMEM((CHUNK,), jnp.float32),
        ]
    scratch.append(pltpu.VMEM_SHARED((NP, D), jnp.float32))
    if with_cnt:
        scratch.append(pltpu.VMEM_SHARED((NP,), jnp.float32))
    scratch += [pltpu.SemaphoreType.DMA, pltpu.SemaphoreType.DMA]

    return pl.kernel(
        functools.partial(_sc_agg_body, with_cnt),
        out_type=out_type,
        mesh=mesh,
        scratch_types=scratch,
    )


R = 1024
_G = (N + R - 1) // R


def _combine_mean(agg_ref, cnt_ref):
    i = pl.program_id(0)
    s = agg_ref[0] + agg_ref[1]
    c = cnt_ref[0, pl.ds(i * R, R)] + cnt_ref[1, pl.ds(i * R, R)]
    inv = 1.0 / jnp.clip(c, 1.0, None)
    return s * inv[:, None]


def _layer_out(agg_ref, cnt_ref, h_ref, wl_ref, wr_ref, bl_ref):
    mean = _combine_mean(agg_ref, cnt_ref)
    return (jnp.dot(mean, wl_ref[...], preferred_element_type=jnp.float32)
            + jnp.dot(h_ref[...], wr_ref[...], preferred_element_type=jnp.float32)
            + bl_ref[...])


def _tc_mid_body(agg_ref, cnt_ref, h_ref, wl_ref, wr_ref, bl_ref, g_ref, b_ref,
                 h_out):
    pre = _layer_out(agg_ref, cnt_ref, h_ref, wl_ref, wr_ref, bl_ref)
    h = jnp.maximum(pre, 0.0)
    mu = jnp.mean(h, axis=-1, keepdims=True)
    var = jnp.mean((h - mu) ** 2, axis=-1, keepdims=True)
    h_out[...] = (h - mu) * lax.rsqrt(var + EPS) * g_ref[...] + b_ref[...]


def _tc_post_body(agg_ref, cnt_ref, h_ref, wl_ref, wr_ref, bl_ref, out_ref):
    out_ref[...] = _layer_out(agg_ref, cnt_ref, h_ref, wl_ref, wr_ref, bl_ref)


_row_spec = pl.BlockSpec((R, D), lambda i: (i, 0))
_w_spec = pl.BlockSpec((D, D), lambda i: (0, 0))
_b_spec = pl.BlockSpec((1, D), lambda i: (0, 0))
_agg_spec = pl.BlockSpec((NC, R, D), lambda i: (0, i, 0))
_cnt_spec = pl.BlockSpec((NC, NP), lambda i: (0, 0))

_tc_mid = pl.pallas_call(
    _tc_mid_body,
    grid=(_G,),
    in_specs=[_agg_spec, _cnt_spec, _row_spec, _w_spec, _w_spec,
              _b_spec, _b_spec, _b_spec],
    out_specs=_row_spec,
    out_shape=jax.ShapeDtypeStruct((N, D), jnp.float32),
)

_tc_post = pl.pallas_call(
    _tc_post_body,
    grid=(_G,),
    in_specs=[_agg_spec, _cnt_spec, _row_spec, _w_spec, _w_spec, _b_spec],
    out_specs=_row_spec,
    out_shape=jax.ShapeDtypeStruct((N, D), jnp.float32),
)


def kernel(x, edge_index, Wl0, bl0, Wr0, Wl1, bl1, Wr1, Wl2, bl2, Wr2, ln_g, ln_b):
    pad_iota = jnp.arange(EP - E, dtype=jnp.int32)
    src = jnp.concatenate([edge_index[0], pad_iota]).reshape(EROWS, CHUNK)
    dst = jnp.concatenate([edge_index[1],
                           N + (pad_iota & 127)]).reshape(EROWS, CHUNK)
    bl0_2 = bl0.reshape(1, D)
    bl1_2 = bl1.reshape(1, D)
    bl2_2 = bl2.reshape(1, D)
    g2 = ln_g.reshape(1, D)
    b2 = ln_b.reshape(1, D)

    sc_agg_cnt = _make_sc_agg(True)
    sc_agg = _make_sc_agg(False)

    agg0, cnt = sc_agg_cnt(x, src, dst)
    h1 = _tc_mid(agg0, cnt, x, Wl0.T, Wr0.T, bl0_2, g2, b2)
    (agg1,) = sc_agg(h1, src, dst)
    h2 = _tc_mid(agg1, cnt, h1, Wl1.T, Wr1.T, bl1_2, g2, b2)
    (agg2,) = sc_agg(h2, src, dst)
    return _tc_post(agg2, cnt, h2, Wl2.T, Wr2.T, bl2_2)
